# baseline (speedup 1.0000x reference)
import functools

import jax
import jax.numpy as jnp
from jax import lax
from jax.experimental import pallas as pl
from jax.experimental.pallas import tpu as pltpu
from jax.experimental.pallas import tpu_sc as plsc

NND = 10000
NED = 160000
DIN = 256
DH = 512
NCLS = 16
NG = 64

NPAD = 10240
E2 = 163840
NBALL = E2 // 128
CH = DH // 128
BR = 2560
BR2 = 2048
STRIPE = NPAD // 16
F32 = jnp.float32


def _sc_mesh():
    return plsc.VectorSubcoreMesh(core_axis_name="c", subcore_axis_name="s")


@functools.lru_cache(maxsize=None)
def _deg_kernel():
    nbt = (E2 // 128) // 32

    @functools.partial(
        pl.kernel, mesh=_sc_mesh(),
        out_type=jax.ShapeDtypeStruct((2 * NPAD, 128), F32),
        scratch_types=[
            pltpu.VMEM((nbt, 128), jnp.int32),
            pltpu.VMEM((128, 128), F32),
            pltpu.VMEM((16, 128), F32),
            pltpu.VMEM_SHARED((NPAD, 128), F32),
            pltpu.SemaphoreType.DMA,
        ],
    )
    def k(dstm, out, didx, ones, zbuf, acc, sem):
        core = lax.axis_index("c")
        sid = lax.axis_index("s")
        wid = core * 16 + sid
        row0 = sid * STRIPE

        onev = jnp.ones((16,), F32)
        zv = jnp.zeros((16,), F32)

        def fill(i, _):
            for j2 in range(8):
                ones[i, pl.ds(j2 * 16, 16)] = onev
            return 0

        lax.fori_loop(0, 128, fill, 0)

        def fillz(i, _):
            for j2 in range(8):
                zbuf[i, pl.ds(j2 * 16, 16)] = zv
            return 0

        lax.fori_loop(0, 16, fillz, 0)

        pltpu.sync_copy(dstm.at[pl.ds(wid * nbt, nbt)], didx)
        for b2 in range(STRIPE // 16):
            pltpu.sync_copy(zbuf, acc.at[pl.ds(row0 + b2 * 16, 16)])
        plsc.subcore_barrier()

        def body(b, _):
            pltpu.sync_copy(ones, acc.at[didx.at[b]], add=True)
            return 0

        lax.fori_loop(0, nbt, body, 0)
        plsc.subcore_barrier()
        pltpu.sync_copy(acc.at[pl.ds(row0, STRIPE)],
                        out.at[pl.ds(core * NPAD + row0, STRIPE)])

    return k


@functools.lru_cache(maxsize=None)
def _seg_kernel(split_edges: bool):
    EB = E2 // 128
    nbt = EB // (32 if split_edges else 16)
    nsteps = nbt // 2
    out_rows = (2 if split_edges else CH) * NPAD

    @functools.partial(
        pl.kernel, mesh=_sc_mesh(),
        out_type=jax.ShapeDtypeStruct((out_rows, 128), F32),
        scratch_types=[
            pltpu.VMEM((nbt, 128), jnp.int32),
            pltpu.VMEM((1, 128), jnp.int32),
            pltpu.VMEM((1, 128), jnp.int32),
            pltpu.VMEM((128, 128), F32),
            pltpu.VMEM((128, 128), F32),
            pltpu.VMEM((16, 128), F32),
            pltpu.VMEM_SHARED((NPAD, 128), F32),
            pltpu.SemaphoreType.DMA,
            pltpu.SemaphoreType.DMA,
            pltpu.SemaphoreType.DMA,
            pltpu.SemaphoreType.DMA,
            pltpu.SemaphoreType.DMA,
            pltpu.SemaphoreType.DMA,
        ],
    )
    def k(tbl, srcm, dstm, out, sidx, rida, ridb, rowsa, rowsb, zbuf, acc,
          sema, semb, semia, semib, semsa, semsb):
        core = lax.axis_index("c")
        sid = lax.axis_index("s")
        row0 = sid * STRIPE

        zv = jnp.zeros((16,), F32)

        def zrow(i, _):
            for j2 in range(8):
                zbuf[i, pl.ds(j2 * 16, 16)] = zv
            return 0

        lax.fori_loop(0, 16, zrow, 0)

        if split_edges:
            erow0 = (core * 16 + sid) * nbt
            chunks = (0,)
        else:
            erow0 = sid * nbt
            chunks = (0, 1)

        for j in chunks:
            if split_edges:
                ochunk = core
                base = erow0
            else:
                ochunk = core * 2 + j
                base = ochunk * EB + erow0
            pltpu.sync_copy(srcm.at[pl.ds(base, nbt)], sidx)
            for b2 in range(STRIPE // 16):
                pltpu.sync_copy(zbuf, acc.at[pl.ds(row0 + b2 * 16, 16)])
            plsc.subcore_barrier()

            wga = pltpu.make_async_copy(tbl.at[sidx.at[0]], rowsa, sema)
            wgb = pltpu.make_async_copy(tbl.at[sidx.at[0]], rowsb, semb)
            wia = pltpu.make_async_copy(dstm.at[pl.ds(erow0, 1)],
                                        rida, semia)
            wib = pltpu.make_async_copy(dstm.at[pl.ds(erow0, 1)],
                                        ridb, semib)
            wsa = pltpu.make_async_copy(rowsa, acc.at[rida.at[0]], semsa)
            wsb = pltpu.make_async_copy(rowsb, acc.at[ridb.at[0]], semsb)

            pltpu.async_copy(dstm.at[pl.ds(erow0, 1)], rida, semia)
            pltpu.async_copy(dstm.at[pl.ds(erow0 + 1, 1)], ridb, semib)
            pltpu.async_copy(tbl.at[sidx.at[0]], rowsa, sema)

            def body(g, _):
                b0 = 2 * g

                @pl.when(g > 0)
                def _():
                    wsb.wait()
                    pltpu.async_copy(
                        dstm.at[pl.ds(erow0 + b0 + 1, 1)], ridb, semib)

                pltpu.async_copy(tbl.at[sidx.at[b0 + 1]], rowsb, semb)
                wga.wait()
                wia.wait()
                pltpu.async_copy(rowsa, acc.at[rida.at[0]], semsa,
                                 add=True)
                wsa.wait()
                pltpu.async_copy(
                    dstm.at[pl.ds(erow0 + lax.rem(b0 + 2, nbt), 1)],
                    rida, semia)
                pltpu.async_copy(tbl.at[sidx.at[lax.rem(b0 + 2, nbt)]],
                                 rowsa, sema)
                wgb.wait()
                wib.wait()
                pltpu.async_copy(rowsb, acc.at[ridb.at[0]], semsb,
                                 add=True)
                return 0

            lax.fori_loop(0, nsteps, body, 0)
            wga.wait()
            wia.wait()
            wsb.wait()
            plsc.subcore_barrier()
            pltpu.sync_copy(acc.at[pl.ds(row0, STRIPE)],
                            out.at[pl.ds(ochunk * NPAD + row0, STRIPE)])
            if j != chunks[-1]:
                plsc.subcore_barrier()

    return k


def _xw_call(xp, W0):
    def body(x_ref, w_ref, o_ref):
        o_ref[...] = jnp.dot(x_ref[...], w_ref[...],
                             preferred_element_type=F32)

    return pl.pallas_call(
        body,
        grid=(NPAD // BR,),
        in_specs=[
            pl.BlockSpec((BR, DIN), lambda i: (i, 0)),
            pl.BlockSpec((DIN, DH), lambda i: (0, 0)),
        ],
        out_specs=pl.BlockSpec((BR, DH), lambda i: (i, 0)),
        out_shape=jax.ShapeDtypeStruct((NPAD, DH), F32),
        compiler_params=pltpu.CompilerParams(
            dimension_semantics=("parallel",)),
    )(xp, W0)


def _mm0_call(xw, degp):
    def body(x_ref, d_ref, o_ref, dv_ref):
        dv = lax.rsqrt(d_ref[0, :, 0:1] + d_ref[1, :, 0:1] + 1.0)
        dv_ref[...] = dv
        p = x_ref[...]
        for c2 in range(CH):
            o_ref[c2] = dv * p[:, c2 * 128:(c2 + 1) * 128]

    return pl.pallas_call(
        body,
        grid=(NPAD // BR,),
        in_specs=[
            pl.BlockSpec((BR, DH), lambda i: (i, 0)),
            pl.BlockSpec((2, BR, 128), lambda i: (0, i, 0)),
        ],
        out_specs=[
            pl.BlockSpec((CH, BR, 128), lambda i: (0, i, 0)),
            pl.BlockSpec((BR, 1), lambda i: (i, 0)),
        ],
        out_shape=[
            jax.ShapeDtypeStruct((CH, NPAD, 128), F32),
            jax.ShapeDtypeStruct((NPAD, 1), F32),
        ],
        compiler_params=pltpu.CompilerParams(
            dimension_semantics=("parallel",)),
    )(xw, degp)


def _mm_mid_call(S, hws, b4, W, dinv):
    def body(s_ref, h_ref, b_ref, w_ref, dv_ref, o_ref):
        kk = pl.program_id(1)
        dv = dv_ref[...]
        a = jnp.maximum(dv * (s_ref[0] + h_ref[0]) + b_ref[0], 0.0)
        p = jnp.dot(a, w_ref[...], preferred_element_type=F32)

        @pl.when(kk == 0)
        def _():
            for c2 in range(CH):
                o_ref[c2] = p[:, c2 * 128:(c2 + 1) * 128]

        @pl.when(kk > 0)
        def _():
            for c2 in range(CH):
                o_ref[c2] = o_ref[c2] + p[:, c2 * 128:(c2 + 1) * 128]

        @pl.when(kk == CH - 1)
        def _():
            for c2 in range(CH):
                o_ref[c2] = dv * o_ref[c2]

    return pl.pallas_call(
        body,
        grid=(NPAD // BR, CH),
        in_specs=[
            pl.BlockSpec((1, BR, 128), lambda i, k: (k, i, 0)),
            pl.BlockSpec((1, BR, 128), lambda i, k: (k, i, 0)),
            pl.BlockSpec((1, 1, 128), lambda i, k: (k, 0, 0)),
            pl.BlockSpec((128, DH), lambda i, k: (k, 0)),
            pl.BlockSpec((BR, 1), lambda i, k: (i, 0)),
        ],
        out_specs=pl.BlockSpec((CH, BR, 128), lambda i, k: (0, i, 0)),
        out_shape=jax.ShapeDtypeStruct((CH, NPAD, 128), F32),
        compiler_params=pltpu.CompilerParams(
            dimension_semantics=("parallel", "arbitrary")),
    )(S, hws, b4, W, dinv)


def _mm_out_call(S, hws, b4, W4, dinv):
    def body(s_ref, h_ref, b_ref, w_ref, dv_ref, o_ref):
        kk = pl.program_id(1)
        dv = dv_ref[...]
        a = jnp.maximum(dv * (s_ref[0] + h_ref[0]) + b_ref[0], 0.0)
        p = jnp.dot(a, w_ref[0], preferred_element_type=F32)

        @pl.when(kk == 0)
        def _():
            o_ref[...] = p

        @pl.when(kk > 0)
        def _():
            o_ref[...] = o_ref[...] + p

        @pl.when(kk == CH - 1)
        def _():
            o_ref[...] = dv * o_ref[...]

    return pl.pallas_call(
        body,
        grid=(NPAD // BR, CH),
        in_specs=[
            pl.BlockSpec((1, BR, 128), lambda i, k: (k, i, 0)),
            pl.BlockSpec((1, BR, 128), lambda i, k: (k, i, 0)),
            pl.BlockSpec((1, 1, 128), lambda i, k: (k, 0, 0)),
            pl.BlockSpec((1, 128, 128), lambda i, k: (k, 0, 0)),
            pl.BlockSpec((BR, 1), lambda i, k: (i, 0)),
        ],
        out_specs=pl.BlockSpec((BR, 128), lambda i, k: (i, 0)),
        out_shape=jax.ShapeDtypeStruct((NPAD, 128), F32),
        compiler_params=pltpu.CompilerParams(
            dimension_semantics=("parallel", "arbitrary")),
    )(S, hws, b4, W4, dinv)


def _pool_call(Sf, hw, dinv, batch2, boutp):
    nt = NPAD // BR2

    def body(s_ref, h_ref, dv_ref, b_ref, bo_ref, o_ref, accs, accc):
        t = pl.program_id(0)
        hout = dv_ref[...] * (s_ref[0] + s_ref[1] + h_ref[...])
        bb = b_ref[...][:, 0]
        mask = (lax.broadcasted_iota(jnp.int32, (NG, BR2), 0)
                == bb[None, :]).astype(F32)
        ps = jnp.dot(mask, hout, preferred_element_type=F32)
        pc = jnp.broadcast_to(jnp.sum(mask, axis=1, keepdims=True), (NG, 128))

        @pl.when(t == 0)
        def _():
            accs[...] = ps
            accc[...] = pc

        @pl.when(t > 0)
        def _():
            accs[...] = accs[...] + ps
            accc[...] = accc[...] + pc

        @pl.when(t == nt - 1)
        def _():
            o_ref[...] = accs[...] / jnp.maximum(accc[...], 1.0) + bo_ref[...]

    return pl.pallas_call(
        body,
        grid=(nt,),
        in_specs=[
            pl.BlockSpec((2, BR2, 128), lambda t: (0, t, 0)),
            pl.BlockSpec((BR2, 128), lambda t: (t, 0)),
            pl.BlockSpec((BR2, 1), lambda t: (t, 0)),
            pl.BlockSpec((BR2, 1), lambda t: (t, 0)),
            pl.BlockSpec((1, 128), lambda t: (0, 0)),
        ],
        out_specs=pl.BlockSpec((NG, 128), lambda t: (0, 0)),
        out_shape=jax.ShapeDtypeStruct((NG, 128), F32),
        scratch_shapes=[
            pltpu.VMEM((NG, 128), F32),
            pltpu.VMEM((NG, 128), F32),
        ],
        compiler_params=pltpu.CompilerParams(
            dimension_semantics=("arbitrary",)),
    )(Sf, hw, dinv, batch2, boutp)


def kernel(x, edge_index, batch, W0, b0, W1, b1, W2, b2, Wout, bout):
    src = edge_index[0]
    dst = edge_index[1]
    padi = NND + (jnp.arange(E2 - NED, dtype=jnp.int32) % 128)
    srcp = jnp.concatenate([src, padi])
    dstp = jnp.concatenate([dst, padi])
    EB = E2 // 128
    src2 = srcp.reshape(EB, 128)
    dst2 = dstp.reshape(EB, 128)
    src4 = (srcp[None, :]
            + (jnp.arange(CH, dtype=jnp.int32) * NPAD)[:, None]
            ).reshape(CH * EB, 128)
    xp = jnp.pad(x, ((0, NPAD - NND), (0, 0)))
    batch2 = jnp.pad(batch, (0, NPAD - NND),
                     constant_values=NG).reshape(NPAD, 1)
    b0r = b0.reshape(CH, 1, 128)
    b1r = b1.reshape(CH, 1, 128)
    b2r = b2.reshape(CH, 1, 128)
    Wout4 = jnp.pad(Wout, ((0, 0), (0, 128 - NCLS))).reshape(CH, 128, 128)
    boutp = jnp.pad(bout, (0, 128 - NCLS)).reshape(1, 128)

    xw = _xw_call(xp, W0)
    degp = _deg_kernel()(dst2).reshape(2, NPAD, 128)

    seg = _seg_kernel(False)
    hws0, dinv = _mm0_call(xw, degp)
    S0 = seg(hws0.reshape(CH * NPAD, 128), src4, dst2).reshape(CH, NPAD, 128)
    hws1 = _mm_mid_call(S0, hws0, b0r, W1, dinv)
    S1 = seg(hws1.reshape(CH * NPAD, 128), src4, dst2).reshape(CH, NPAD, 128)
    hws2 = _mm_mid_call(S1, hws1, b1r, W2, dinv)
    S2 = seg(hws2.reshape(CH * NPAD, 128), src4, dst2).reshape(CH, NPAD, 128)
    hwo = _mm_out_call(S2, hws2, b2r, Wout4, dinv)
    Sf = _seg_kernel(True)(hwo, src2, dst2).reshape(2, NPAD, 128)
    Hg = _pool_call(Sf, hwo, dinv, batch2, boutp)
    return Hg[:, :NCLS]


# speedup vs baseline: 1.0036x; 1.0036x over previous
import functools

import jax
import jax.numpy as jnp
from jax import lax
from jax.experimental import pallas as pl
from jax.experimental.pallas import tpu as pltpu
from jax.experimental.pallas import tpu_sc as plsc

NND = 10000
NED = 160000
DIN = 256
DH = 512
NCLS = 16
NG = 64

NPAD = 10240
E2 = 163840
NBALL = E2 // 128
CH = DH // 128
BR = 2560
BR2 = 2048
STRIPE = NPAD // 16
F32 = jnp.float32


def _sc_mesh():
    return plsc.VectorSubcoreMesh(core_axis_name="c", subcore_axis_name="s")


@functools.lru_cache(maxsize=None)
def _deg_kernel():
    nbt = (E2 // 128) // 32

    @functools.partial(
        pl.kernel, mesh=_sc_mesh(),
        out_type=jax.ShapeDtypeStruct((2 * NPAD, 128), F32),
        scratch_types=[
            pltpu.VMEM((nbt, 128), jnp.int32),
            pltpu.VMEM((128, 128), F32),
            pltpu.VMEM((16, 128), F32),
            pltpu.VMEM_SHARED((NPAD, 128), F32),
            pltpu.SemaphoreType.DMA,
        ],
    )
    def k(dstm, out, didx, ones, zbuf, acc, sem):
        core = lax.axis_index("c")
        sid = lax.axis_index("s")
        wid = core * 16 + sid
        row0 = sid * STRIPE

        onev = jnp.ones((16,), F32)
        zv = jnp.zeros((16,), F32)

        def fill(i, _):
            for j2 in range(8):
                ones[i, pl.ds(j2 * 16, 16)] = onev
            return 0

        lax.fori_loop(0, 128, fill, 0)

        def fillz(i, _):
            for j2 in range(8):
                zbuf[i, pl.ds(j2 * 16, 16)] = zv
            return 0

        lax.fori_loop(0, 16, fillz, 0)

        pltpu.sync_copy(dstm.at[pl.ds(wid * nbt, nbt)], didx)
        for b2 in range(STRIPE // 16):
            pltpu.sync_copy(zbuf, acc.at[pl.ds(row0 + b2 * 16, 16)])
        plsc.subcore_barrier()

        def body(b, _):
            pltpu.sync_copy(ones, acc.at[didx.at[b]], add=True)
            return 0

        lax.fori_loop(0, nbt, body, 0)
        plsc.subcore_barrier()
        pltpu.sync_copy(acc.at[pl.ds(row0, STRIPE)],
                        out.at[pl.ds(core * NPAD + row0, STRIPE)])

    return k


@functools.lru_cache(maxsize=None)
def _seg_kernel(split_edges: bool):
    EB = E2 // 128
    nbt = EB // (32 if split_edges else 16)
    nsteps = nbt // 2
    out_rows = (2 if split_edges else CH) * NPAD

    @functools.partial(
        pl.kernel, mesh=_sc_mesh(),
        out_type=jax.ShapeDtypeStruct((out_rows, 128), F32),
        scratch_types=[
            pltpu.VMEM((nbt, 128), jnp.int32),
            pltpu.VMEM((1, 128), jnp.int32),
            pltpu.VMEM((1, 128), jnp.int32),
            pltpu.VMEM((128, 128), F32),
            pltpu.VMEM((128, 128), F32),
            pltpu.VMEM((16, 128), F32),
            pltpu.VMEM_SHARED((NPAD, 128), F32),
            pltpu.SemaphoreType.DMA,
            pltpu.SemaphoreType.DMA,
            pltpu.SemaphoreType.DMA,
            pltpu.SemaphoreType.DMA,
            pltpu.SemaphoreType.DMA,
            pltpu.SemaphoreType.DMA,
        ],
    )
    def k(tbl, srcm, dstm, out, sidx, rida, ridb, rowsa, rowsb, zbuf, acc,
          sema, semb, semia, semib, semsa, semsb):
        core = lax.axis_index("c")
        sid = lax.axis_index("s")
        row0 = sid * STRIPE

        zv = jnp.zeros((16,), F32)

        def zrow(i, _):
            for j2 in range(8):
                zbuf[i, pl.ds(j2 * 16, 16)] = zv
            return 0

        lax.fori_loop(0, 16, zrow, 0)

        if split_edges:
            erow0 = (core * 16 + sid) * nbt
            chunks = (0,)
        else:
            erow0 = sid * nbt
            chunks = (0, 1)

        for j in chunks:
            if split_edges:
                ochunk = core
                base = erow0
            else:
                ochunk = core * 2 + j
                base = ochunk * EB + erow0
            pltpu.sync_copy(srcm.at[pl.ds(base, nbt)], sidx)
            for b2 in range(STRIPE // 16):
                pltpu.sync_copy(zbuf, acc.at[pl.ds(row0 + b2 * 16, 16)])
            plsc.subcore_barrier()

            wga = pltpu.make_async_copy(tbl.at[sidx.at[0]], rowsa, sema)
            wgb = pltpu.make_async_copy(tbl.at[sidx.at[0]], rowsb, semb)
            wia = pltpu.make_async_copy(dstm.at[pl.ds(erow0, 1)],
                                        rida, semia)
            wib = pltpu.make_async_copy(dstm.at[pl.ds(erow0, 1)],
                                        ridb, semib)
            wsa = pltpu.make_async_copy(rowsa, acc.at[rida.at[0]], semsa)
            wsb = pltpu.make_async_copy(rowsb, acc.at[ridb.at[0]], semsb)

            pltpu.async_copy(dstm.at[pl.ds(erow0, 1)], rida, semia)
            pltpu.async_copy(dstm.at[pl.ds(erow0 + 1, 1)], ridb, semib)
            pltpu.async_copy(tbl.at[sidx.at[0]], rowsa, sema)

            def body(g, _):
                b0 = 2 * g

                @pl.when(g > 0)
                def _():
                    wsb.wait()
                    pltpu.async_copy(
                        dstm.at[pl.ds(erow0 + b0 + 1, 1)], ridb, semib)

                pltpu.async_copy(tbl.at[sidx.at[b0 + 1]], rowsb, semb)
                wga.wait()
                wia.wait()
                pltpu.async_copy(rowsa, acc.at[rida.at[0]], semsa,
                                 add=True)
                wsa.wait()
                pltpu.async_copy(
                    dstm.at[pl.ds(erow0 + lax.rem(b0 + 2, nbt), 1)],
                    rida, semia)
                pltpu.async_copy(tbl.at[sidx.at[lax.rem(b0 + 2, nbt)]],
                                 rowsa, sema)
                wgb.wait()
                wib.wait()
                pltpu.async_copy(rowsb, acc.at[ridb.at[0]], semsb,
                                 add=True)
                return 0

            lax.fori_loop(0, nsteps, body, 0)
            wga.wait()
            wia.wait()
            wsb.wait()
            plsc.subcore_barrier()
            pltpu.sync_copy(acc.at[pl.ds(row0, STRIPE)],
                            out.at[pl.ds(ochunk * NPAD + row0, STRIPE)])
            if j != chunks[-1]:
                plsc.subcore_barrier()

    return k


def _mm0_call(xp, W0, degp):
    def body(x_ref, w_ref, d_ref, o_ref, dv_ref):
        dv = lax.rsqrt(d_ref[0, :, 0:1] + d_ref[1, :, 0:1] + 1.0)
        dv_ref[...] = dv
        p = jnp.dot(x_ref[...], w_ref[...], preferred_element_type=F32)
        for c2 in range(CH):
            o_ref[c2] = dv * p[:, c2 * 128:(c2 + 1) * 128]

    return pl.pallas_call(
        body,
        grid=(NPAD // BR,),
        in_specs=[
            pl.BlockSpec((BR, DIN), lambda i: (i, 0)),
            pl.BlockSpec((DIN, DH), lambda i: (0, 0)),
            pl.BlockSpec((2, BR, 128), lambda i: (0, i, 0)),
        ],
        out_specs=[
            pl.BlockSpec((CH, BR, 128), lambda i: (0, i, 0)),
            pl.BlockSpec((BR, 1), lambda i: (i, 0)),
        ],
        out_shape=[
            jax.ShapeDtypeStruct((CH, NPAD, 128), F32),
            jax.ShapeDtypeStruct((NPAD, 1), F32),
        ],
        compiler_params=pltpu.CompilerParams(
            dimension_semantics=("parallel",)),
    )(xp, W0, degp)


def _mm_mid_call(S, hws, b4, W, dinv):
    def body(s_ref, h_ref, b_ref, w_ref, dv_ref, o_ref):
        kk = pl.program_id(1)
        dv = dv_ref[...]
        a = jnp.maximum(dv * (s_ref[0] + h_ref[0]) + b_ref[0], 0.0)
        p = jnp.dot(a, w_ref[...], preferred_element_type=F32)

        @pl.when(kk == 0)
        def _():
            for c2 in range(CH):
                o_ref[c2] = p[:, c2 * 128:(c2 + 1) * 128]

        @pl.when(kk > 0)
        def _():
            for c2 in range(CH):
                o_ref[c2] = o_ref[c2] + p[:, c2 * 128:(c2 + 1) * 128]

        @pl.when(kk == CH - 1)
        def _():
            for c2 in range(CH):
                o_ref[c2] = dv * o_ref[c2]

    return pl.pallas_call(
        body,
        grid=(NPAD // BR, CH),
        in_specs=[
            pl.BlockSpec((1, BR, 128), lambda i, k: (k, i, 0)),
            pl.BlockSpec((1, BR, 128), lambda i, k: (k, i, 0)),
            pl.BlockSpec((1, 1, 128), lambda i, k: (k, 0, 0)),
            pl.BlockSpec((128, DH), lambda i, k: (k, 0)),
            pl.BlockSpec((BR, 1), lambda i, k: (i, 0)),
        ],
        out_specs=pl.BlockSpec((CH, BR, 128), lambda i, k: (0, i, 0)),
        out_shape=jax.ShapeDtypeStruct((CH, NPAD, 128), F32),
        compiler_params=pltpu.CompilerParams(
            dimension_semantics=("parallel", "arbitrary")),
    )(S, hws, b4, W, dinv)


def _mm_out_call(S, hws, b4, W4, dinv):
    def body(s_ref, h_ref, b_ref, w_ref, dv_ref, o_ref):
        kk = pl.program_id(1)
        dv = dv_ref[...]
        a = jnp.maximum(dv * (s_ref[0] + h_ref[0]) + b_ref[0], 0.0)
        p = jnp.dot(a, w_ref[0], preferred_element_type=F32)

        @pl.when(kk == 0)
        def _():
            o_ref[...] = p

        @pl.when(kk > 0)
        def _():
            o_ref[...] = o_ref[...] + p

        @pl.when(kk == CH - 1)
        def _():
            o_ref[...] = dv * o_ref[...]

    return pl.pallas_call(
        body,
        grid=(NPAD // BR, CH),
        in_specs=[
            pl.BlockSpec((1, BR, 128), lambda i, k: (k, i, 0)),
            pl.BlockSpec((1, BR, 128), lambda i, k: (k, i, 0)),
            pl.BlockSpec((1, 1, 128), lambda i, k: (k, 0, 0)),
            pl.BlockSpec((1, 128, 128), lambda i, k: (k, 0, 0)),
            pl.BlockSpec((BR, 1), lambda i, k: (i, 0)),
        ],
        out_specs=pl.BlockSpec((BR, 128), lambda i, k: (i, 0)),
        out_shape=jax.ShapeDtypeStruct((NPAD, 128), F32),
        compiler_params=pltpu.CompilerParams(
            dimension_semantics=("parallel", "arbitrary")),
    )(S, hws, b4, W4, dinv)


def _pool_call(Sf, hw, dinv, batch2, boutp):
    nt = NPAD // BR2

    def body(s_ref, h_ref, dv_ref, b_ref, bo_ref, o_ref, accs, accc):
        t = pl.program_id(0)
        hout = dv_ref[...] * (s_ref[0] + s_ref[1] + h_ref[...])
        bb = b_ref[...][:, 0]
        mask = (lax.broadcasted_iota(jnp.int32, (NG, BR2), 0)
                == bb[None, :]).astype(F32)
        ps = jnp.dot(mask, hout, preferred_element_type=F32)
        pc = jnp.broadcast_to(jnp.sum(mask, axis=1, keepdims=True), (NG, 128))

        @pl.when(t == 0)
        def _():
            accs[...] = ps
            accc[...] = pc

        @pl.when(t > 0)
        def _():
            accs[...] = accs[...] + ps
            accc[...] = accc[...] + pc

        @pl.when(t == nt - 1)
        def _():
            o_ref[...] = accs[...] / jnp.maximum(accc[...], 1.0) + bo_ref[...]

    return pl.pallas_call(
        body,
        grid=(nt,),
        in_specs=[
            pl.BlockSpec((2, BR2, 128), lambda t: (0, t, 0)),
            pl.BlockSpec((BR2, 128), lambda t: (t, 0)),
            pl.BlockSpec((BR2, 1), lambda t: (t, 0)),
            pl.BlockSpec((BR2, 1), lambda t: (t, 0)),
            pl.BlockSpec((1, 128), lambda t: (0, 0)),
        ],
        out_specs=pl.BlockSpec((NG, 128), lambda t: (0, 0)),
        out_shape=jax.ShapeDtypeStruct((NG, 128), F32),
        scratch_shapes=[
            pltpu.VMEM((NG, 128), F32),
            pltpu.VMEM((NG, 128), F32),
        ],
        compiler_params=pltpu.CompilerParams(
            dimension_semantics=("arbitrary",)),
    )(Sf, hw, dinv, batch2, boutp)


def kernel(x, edge_index, batch, W0, b0, W1, b1, W2, b2, Wout, bout):
    src = edge_index[0]
    dst = edge_index[1]
    padi = NND + (jnp.arange(E2 - NED, dtype=jnp.int32) % 128)
    srcp = jnp.concatenate([src, padi])
    dstp = jnp.concatenate([dst, padi])
    EB = E2 // 128
    src2 = srcp.reshape(EB, 128)
    dst2 = dstp.reshape(EB, 128)
    src4 = (srcp[None, :]
            + (jnp.arange(CH, dtype=jnp.int32) * NPAD)[:, None]
            ).reshape(CH * EB, 128)
    xp = jnp.pad(x, ((0, NPAD - NND), (0, 0)))
    batch2 = jnp.pad(batch, (0, NPAD - NND),
                     constant_values=NG).reshape(NPAD, 1)
    b0r = b0.reshape(CH, 1, 128)
    b1r = b1.reshape(CH, 1, 128)
    b2r = b2.reshape(CH, 1, 128)
    Wout4 = jnp.pad(Wout, ((0, 0), (0, 128 - NCLS))).reshape(CH, 128, 128)
    boutp = jnp.pad(bout, (0, 128 - NCLS)).reshape(1, 128)

    degp = _deg_kernel()(dst2).reshape(2, NPAD, 128)

    seg = _seg_kernel(False)
    hws0, dinv = _mm0_call(xp, W0, degp)
    S0 = seg(hws0.reshape(CH * NPAD, 128), src4, dst2).reshape(CH, NPAD, 128)
    hws1 = _mm_mid_call(S0, hws0, b0r, W1, dinv)
    S1 = seg(hws1.reshape(CH * NPAD, 128), src4, dst2).reshape(CH, NPAD, 128)
    hws2 = _mm_mid_call(S1, hws1, b1r, W2, dinv)
    S2 = seg(hws2.reshape(CH * NPAD, 128), src4, dst2).reshape(CH, NPAD, 128)
    hwo = _mm_out_call(S2, hws2, b2r, Wout4, dinv)
    Sf = _seg_kernel(True)(hwo, src2, dst2).reshape(2, NPAD, 128)
    Hg = _pool_call(Sf, hwo, dinv, batch2, boutp)
    return Hg[:, :NCLS]


# speedup vs baseline: 1.0173x; 1.0137x over previous
import functools

import jax
import jax.numpy as jnp
from jax import lax
from jax.experimental import pallas as pl
from jax.experimental.pallas import tpu as pltpu
from jax.experimental.pallas import tpu_sc as plsc

NND = 10000
NED = 160000
DIN = 256
DH = 512
NCLS = 16
NG = 64

NPAD = 10240
E2 = 163840
NBALL = E2 // 128
CH = DH // 128
BR = 5120
BR2 = 2048
STRIPE = NPAD // 16
F32 = jnp.float32


def _sc_mesh():
    return plsc.VectorSubcoreMesh(core_axis_name="c", subcore_axis_name="s")


@functools.lru_cache(maxsize=None)
def _deg_kernel():
    nbt = (E2 // 128) // 32

    @functools.partial(
        pl.kernel, mesh=_sc_mesh(),
        out_type=jax.ShapeDtypeStruct((2 * NPAD, 128), F32),
        scratch_types=[
            pltpu.VMEM((nbt, 128), jnp.int32),
            pltpu.VMEM((128, 128), F32),
            pltpu.VMEM((16, 128), F32),
            pltpu.VMEM_SHARED((NPAD, 128), F32),
            pltpu.SemaphoreType.DMA,
        ],
    )
    def k(dstm, out, didx, ones, zbuf, acc, sem):
        core = lax.axis_index("c")
        sid = lax.axis_index("s")
        wid = core * 16 + sid
        row0 = sid * STRIPE

        onev = jnp.ones((16,), F32)
        zv = jnp.zeros((16,), F32)

        def fill(i, _):
            for j2 in range(8):
                ones[i, pl.ds(j2 * 16, 16)] = onev
            return 0

        lax.fori_loop(0, 128, fill, 0)

        def fillz(i, _):
            for j2 in range(8):
                zbuf[i, pl.ds(j2 * 16, 16)] = zv
            return 0

        lax.fori_loop(0, 16, fillz, 0)

        pltpu.sync_copy(dstm.at[pl.ds(wid * nbt, nbt)], didx)
        for b2 in range(STRIPE // 16):
            pltpu.sync_copy(zbuf, acc.at[pl.ds(row0 + b2 * 16, 16)])
        plsc.subcore_barrier()

        def body(b, _):
            pltpu.sync_copy(ones, acc.at[didx.at[b]], add=True)
            return 0

        lax.fori_loop(0, nbt, body, 0)
        plsc.subcore_barrier()
        pltpu.sync_copy(acc.at[pl.ds(row0, STRIPE)],
                        out.at[pl.ds(core * NPAD + row0, STRIPE)])

    return k


@functools.lru_cache(maxsize=None)
def _seg_kernel(split_edges: bool):
    EB = E2 // 128
    nbt = EB // (32 if split_edges else 16)
    nsteps = nbt // 2
    out_rows = (2 if split_edges else CH) * NPAD

    @functools.partial(
        pl.kernel, mesh=_sc_mesh(),
        out_type=jax.ShapeDtypeStruct((out_rows, 128), F32),
        scratch_types=[
            pltpu.VMEM((nbt, 128), jnp.int32),
            pltpu.VMEM((1, 128), jnp.int32),
            pltpu.VMEM((1, 128), jnp.int32),
            pltpu.VMEM((128, 128), F32),
            pltpu.VMEM((128, 128), F32),
            pltpu.VMEM((16, 128), F32),
            pltpu.VMEM_SHARED((NPAD, 128), F32),
            pltpu.SemaphoreType.DMA,
            pltpu.SemaphoreType.DMA,
            pltpu.SemaphoreType.DMA,
            pltpu.SemaphoreType.DMA,
            pltpu.SemaphoreType.DMA,
            pltpu.SemaphoreType.DMA,
        ],
    )
    def k(tbl, srcm, dstm, out, sidx, rida, ridb, rowsa, rowsb, zbuf, acc,
          sema, semb, semia, semib, semsa, semsb):
        core = lax.axis_index("c")
        sid = lax.axis_index("s")
        row0 = sid * STRIPE

        zv = jnp.zeros((16,), F32)

        def zrow(i, _):
            for j2 in range(8):
                zbuf[i, pl.ds(j2 * 16, 16)] = zv
            return 0

        lax.fori_loop(0, 16, zrow, 0)

        if split_edges:
            erow0 = (core * 16 + sid) * nbt
            chunks = (0,)
        else:
            erow0 = sid * nbt
            chunks = (0, 1)

        for j in chunks:
            if split_edges:
                ochunk = core
                base = erow0
            else:
                ochunk = core * 2 + j
                base = ochunk * EB + erow0
            pltpu.sync_copy(srcm.at[pl.ds(base, nbt)], sidx)
            for b2 in range(STRIPE // 16):
                pltpu.sync_copy(zbuf, acc.at[pl.ds(row0 + b2 * 16, 16)])
            plsc.subcore_barrier()

            wga = pltpu.make_async_copy(tbl.at[sidx.at[0]], rowsa, sema)
            wgb = pltpu.make_async_copy(tbl.at[sidx.at[0]], rowsb, semb)
            wia = pltpu.make_async_copy(dstm.at[pl.ds(erow0, 1)],
                                        rida, semia)
            wib = pltpu.make_async_copy(dstm.at[pl.ds(erow0, 1)],
                                        ridb, semib)
            wsa = pltpu.make_async_copy(rowsa, acc.at[rida.at[0]], semsa)
            wsb = pltpu.make_async_copy(rowsb, acc.at[ridb.at[0]], semsb)

            pltpu.async_copy(dstm.at[pl.ds(erow0, 1)], rida, semia)
            pltpu.async_copy(dstm.at[pl.ds(erow0 + 1, 1)], ridb, semib)
            pltpu.async_copy(tbl.at[sidx.at[0]], rowsa, sema)

            def body(g, _):
                b0 = 2 * g

                @pl.when(g > 0)
                def _():
                    wsb.wait()
                    pltpu.async_copy(
                        dstm.at[pl.ds(erow0 + b0 + 1, 1)], ridb, semib)

                pltpu.async_copy(tbl.at[sidx.at[b0 + 1]], rowsb, semb)
                wga.wait()
                wia.wait()
                pltpu.async_copy(rowsa, acc.at[rida.at[0]], semsa,
                                 add=True)
                wsa.wait()
                pltpu.async_copy(
                    dstm.at[pl.ds(erow0 + lax.rem(b0 + 2, nbt), 1)],
                    rida, semia)
                pltpu.async_copy(tbl.at[sidx.at[lax.rem(b0 + 2, nbt)]],
                                 rowsa, sema)
                wgb.wait()
                wib.wait()
                pltpu.async_copy(rowsb, acc.at[ridb.at[0]], semsb,
                                 add=True)
                return 0

            lax.fori_loop(0, nsteps, body, 0)
            wga.wait()
            wia.wait()
            wsb.wait()
            plsc.subcore_barrier()
            pltpu.sync_copy(acc.at[pl.ds(row0, STRIPE)],
                            out.at[pl.ds(ochunk * NPAD + row0, STRIPE)])
            if j != chunks[-1]:
                plsc.subcore_barrier()

    return k


def _mm0_call(xp, W0, degp):
    def body(x_ref, w_ref, d_ref, o_ref, dv_ref):
        dv = lax.rsqrt(d_ref[0, :, 0:1] + d_ref[1, :, 0:1] + 1.0)
        dv_ref[...] = dv
        p = jnp.dot(x_ref[...], w_ref[...], preferred_element_type=F32)
        for c2 in range(CH):
            o_ref[c2] = dv * p[:, c2 * 128:(c2 + 1) * 128]

    return pl.pallas_call(
        body,
        grid=(NPAD // BR,),
        in_specs=[
            pl.BlockSpec((BR, DIN), lambda i: (i, 0)),
            pl.BlockSpec((DIN, DH), lambda i: (0, 0)),
            pl.BlockSpec((2, BR, 128), lambda i: (0, i, 0)),
        ],
        out_specs=[
            pl.BlockSpec((CH, BR, 128), lambda i: (0, i, 0)),
            pl.BlockSpec((BR, 1), lambda i: (i, 0)),
        ],
        out_shape=[
            jax.ShapeDtypeStruct((CH, NPAD, 128), F32),
            jax.ShapeDtypeStruct((NPAD, 1), F32),
        ],
        compiler_params=pltpu.CompilerParams(
            dimension_semantics=("parallel",)),
    )(xp, W0, degp)


def _mm_mid_call(S, hws, b4, W, dinv):
    def body(s_ref, h_ref, b_ref, w_ref, dv_ref, o_ref):
        kk = pl.program_id(1)
        dv = dv_ref[...]
        a = jnp.maximum(dv * (s_ref[0] + h_ref[0]) + b_ref[0], 0.0)
        p = jnp.dot(a, w_ref[...], preferred_element_type=F32)

        @pl.when(kk == 0)
        def _():
            for c2 in range(CH):
                o_ref[c2] = p[:, c2 * 128:(c2 + 1) * 128]

        @pl.when(kk > 0)
        def _():
            for c2 in range(CH):
                o_ref[c2] = o_ref[c2] + p[:, c2 * 128:(c2 + 1) * 128]

        @pl.when(kk == CH - 1)
        def _():
            for c2 in range(CH):
                o_ref[c2] = dv * o_ref[c2]

    return pl.pallas_call(
        body,
        grid=(NPAD // BR, CH),
        in_specs=[
            pl.BlockSpec((1, BR, 128), lambda i, k: (k, i, 0)),
            pl.BlockSpec((1, BR, 128), lambda i, k: (k, i, 0)),
            pl.BlockSpec((1, 1, 128), lambda i, k: (k, 0, 0)),
            pl.BlockSpec((128, DH), lambda i, k: (k, 0)),
            pl.BlockSpec((BR, 1), lambda i, k: (i, 0)),
        ],
        out_specs=pl.BlockSpec((CH, BR, 128), lambda i, k: (0, i, 0)),
        out_shape=jax.ShapeDtypeStruct((CH, NPAD, 128), F32),
        compiler_params=pltpu.CompilerParams(
            dimension_semantics=("parallel", "arbitrary")),
    )(S, hws, b4, W, dinv)


def _mm_out_call(S, hws, b4, W4, dinv):
    def body(s_ref, h_ref, b_ref, w_ref, dv_ref, o_ref):
        kk = pl.program_id(1)
        dv = dv_ref[...]
        a = jnp.maximum(dv * (s_ref[0] + h_ref[0]) + b_ref[0], 0.0)
        p = jnp.dot(a, w_ref[0], preferred_element_type=F32)

        @pl.when(kk == 0)
        def _():
            o_ref[...] = p

        @pl.when(kk > 0)
        def _():
            o_ref[...] = o_ref[...] + p

        @pl.when(kk == CH - 1)
        def _():
            o_ref[...] = dv * o_ref[...]

    return pl.pallas_call(
        body,
        grid=(NPAD // BR, CH),
        in_specs=[
            pl.BlockSpec((1, BR, 128), lambda i, k: (k, i, 0)),
            pl.BlockSpec((1, BR, 128), lambda i, k: (k, i, 0)),
            pl.BlockSpec((1, 1, 128), lambda i, k: (k, 0, 0)),
            pl.BlockSpec((1, 128, 128), lambda i, k: (k, 0, 0)),
            pl.BlockSpec((BR, 1), lambda i, k: (i, 0)),
        ],
        out_specs=pl.BlockSpec((BR, 128), lambda i, k: (i, 0)),
        out_shape=jax.ShapeDtypeStruct((NPAD, 128), F32),
        compiler_params=pltpu.CompilerParams(
            dimension_semantics=("parallel", "arbitrary")),
    )(S, hws, b4, W4, dinv)


def _pool_call(Sf, hw, dinv, batch2, boutp):
    nt = NPAD // BR2

    def body(s_ref, h_ref, dv_ref, b_ref, bo_ref, o_ref, accs, accc):
        t = pl.program_id(0)
        hout = dv_ref[...] * (s_ref[0] + s_ref[1] + h_ref[...])
        bb = b_ref[...][:, 0]
        mask = (lax.broadcasted_iota(jnp.int32, (NG, BR2), 0)
                == bb[None, :]).astype(F32)
        ps = jnp.dot(mask, hout, preferred_element_type=F32)
        pc = jnp.broadcast_to(jnp.sum(mask, axis=1, keepdims=True), (NG, 128))

        @pl.when(t == 0)
        def _():
            accs[...] = ps
            accc[...] = pc

        @pl.when(t > 0)
        def _():
            accs[...] = accs[...] + ps
            accc[...] = accc[...] + pc

        @pl.when(t == nt - 1)
        def _():
            o_ref[...] = accs[...] / jnp.maximum(accc[...], 1.0) + bo_ref[...]

    return pl.pallas_call(
        body,
        grid=(nt,),
        in_specs=[
            pl.BlockSpec((2, BR2, 128), lambda t: (0, t, 0)),
            pl.BlockSpec((BR2, 128), lambda t: (t, 0)),
            pl.BlockSpec((BR2, 1), lambda t: (t, 0)),
            pl.BlockSpec((BR2, 1), lambda t: (t, 0)),
            pl.BlockSpec((1, 128), lambda t: (0, 0)),
        ],
        out_specs=pl.BlockSpec((NG, 128), lambda t: (0, 0)),
        out_shape=jax.ShapeDtypeStruct((NG, 128), F32),
        scratch_shapes=[
            pltpu.VMEM((NG, 128), F32),
            pltpu.VMEM((NG, 128), F32),
        ],
        compiler_params=pltpu.CompilerParams(
            dimension_semantics=("arbitrary",)),
    )(Sf, hw, dinv, batch2, boutp)


def kernel(x, edge_index, batch, W0, b0, W1, b1, W2, b2, Wout, bout):
    src = edge_index[0]
    dst = edge_index[1]
    padi = NND + (jnp.arange(E2 - NED, dtype=jnp.int32) % 128)
    srcp = jnp.concatenate([src, padi])
    dstp = jnp.concatenate([dst, padi])
    EB = E2 // 128
    src2 = srcp.reshape(EB, 128)
    dst2 = dstp.reshape(EB, 128)
    src4 = (srcp[None, :]
            + (jnp.arange(CH, dtype=jnp.int32) * NPAD)[:, None]
            ).reshape(CH * EB, 128)
    xp = jnp.pad(x, ((0, NPAD - NND), (0, 0)))
    batch2 = jnp.pad(batch, (0, NPAD - NND),
                     constant_values=NG).reshape(NPAD, 1)
    b0r = b0.reshape(CH, 1, 128)
    b1r = b1.reshape(CH, 1, 128)
    b2r = b2.reshape(CH, 1, 128)
    Wout4 = jnp.pad(Wout, ((0, 0), (0, 128 - NCLS))).reshape(CH, 128, 128)
    boutp = jnp.pad(bout, (0, 128 - NCLS)).reshape(1, 128)

    degp = _deg_kernel()(dst2).reshape(2, NPAD, 128)

    seg = _seg_kernel(False)
    hws0, dinv = _mm0_call(xp, W0, degp)
    S0 = seg(hws0.reshape(CH * NPAD, 128), src4, dst2).reshape(CH, NPAD, 128)
    hws1 = _mm_mid_call(S0, hws0, b0r, W1, dinv)
    S1 = seg(hws1.reshape(CH * NPAD, 128), src4, dst2).reshape(CH, NPAD, 128)
    hws2 = _mm_mid_call(S1, hws1, b1r, W2, dinv)
    S2 = seg(hws2.reshape(CH * NPAD, 128), src4, dst2).reshape(CH, NPAD, 128)
    hwo = _mm_out_call(S2, hws2, b2r, Wout4, dinv)
    Sf = _seg_kernel(True)(hwo, src2, dst2).reshape(2, NPAD, 128)
    Hg = _pool_call(Sf, hwo, dinv, batch2, boutp)
    return Hg[:, :NCLS]
